# baseline (speedup 1.0000x reference)
import functools

import jax
import jax.numpy as jnp
from jax import lax
from jax.experimental import pallas as pl
from jax.experimental.pallas import tpu as pltpu
from jax.experimental.pallas import tpu_sc as plsc

_B, _S, _H = 4, 8192, 768
_E, _TOPK = 8, 2
_N = _B * _S

_NCHUNK = 2
_NC = _N // _NCHUNK

_BLOCK_T = 2048
_GRID = _NC // _BLOCK_T

_NTILES = 32
_TPT = _NC // _NTILES
_L = 16


def _gate_kernel(x_ref, w_ref, out_ref):
    out_ref[...] = jax.lax.dot_general(
        w_ref[...], x_ref[...], (((1,), (1,)), ((), ())),
        preferred_element_type=jnp.float32)


def _sc_router(lgt_hbm, probs_hbm, wts_hbm, idx_hbm, ewp_hbm, ecp_hbm,
               lg_v, pb_v, wb_v, ib_v, ew_v, ec_v):
    wid = lax.axis_index("s") * 2 + lax.axis_index("c")
    base = wid * _TPT
    pltpu.sync_copy(lgt_hbm.at[:, pl.ds(base, _TPT)], lg_v)

    zero = jnp.zeros((_L,), jnp.float32)

    def chunk(c, carry):
        ews, ecs = carry
        off = c * _L
        ls = [lg_v[e, pl.ds(off, _L)] for e in range(_E)]
        m = ls[0]
        for e in range(1, _E):
            m = jnp.maximum(m, ls[e])
        exs = [jnp.exp(l - m) for l in ls]
        s = exs[0]
        for e in range(1, _E):
            s = s + exs[e]
        inv = 1.0 / s
        ps = [ex * inv for ex in exs]

        best = ps[0]
        bidx = jnp.zeros((_L,), jnp.int32)
        for e in range(1, _E):
            upd = ps[e] > best
            best = jnp.where(upd, ps[e], best)
            bidx = jnp.where(upd, e, bidx)
        second = jnp.full((_L,), -1.0, jnp.float32)
        sidx = jnp.zeros((_L,), jnp.int32)
        for e in range(_E):
            upd = (ps[e] > second) & (bidx != e)
            second = jnp.where(upd, ps[e], second)
            sidx = jnp.where(upd, e, sidx)

        inv2 = 1.0 / (best + second + 1e-8)
        w1 = best * inv2
        w2 = second * inv2

        ii = lax.iota(jnp.int32, _L)
        t8 = (off + ii) * _E
        for e in range(_E):
            plsc.store_scatter(pb_v, [t8 + e], ps[e])
        t2 = (off + ii) * _TOPK
        plsc.store_scatter(wb_v, [t2], w1)
        plsc.store_scatter(wb_v, [t2 + 1], w2)
        plsc.store_scatter(ib_v, [t2], bidx)
        plsc.store_scatter(ib_v, [t2 + 1], sidx)

        new_ews = tuple(
            ews[e] + jnp.where(bidx == e, w1, zero)
            + jnp.where(sidx == e, w2, zero) for e in range(_E))
        new_ecs = tuple(
            ecs[e] + (bidx == e).astype(jnp.float32)
            + (sidx == e).astype(jnp.float32) for e in range(_E))
        return new_ews, new_ecs

    init = (tuple(zero for _ in range(_E)), tuple(zero for _ in range(_E)))
    ews, ecs = lax.fori_loop(0, _TPT // _L, chunk, init)

    ii = lax.iota(jnp.int32, _L)
    ew_row = zero
    ec_row = zero
    for e in range(_E):
        ew_row = jnp.where(ii == e, jnp.sum(ews[e]), ew_row)
        ec_row = jnp.where(ii == e, jnp.sum(ecs[e]), ec_row)
    ew_v[...] = ew_row
    ec_v[...] = ec_row

    pltpu.sync_copy(pb_v, probs_hbm.at[pl.ds(base * _E, _TPT * _E)])
    pltpu.sync_copy(wb_v, wts_hbm.at[pl.ds(base * _TOPK, _TPT * _TOPK)])
    pltpu.sync_copy(ib_v, idx_hbm.at[pl.ds(base * _TOPK, _TPT * _TOPK)])
    pltpu.sync_copy(ew_v, ewp_hbm.at[wid])
    pltpu.sync_copy(ec_v, ecp_hbm.at[wid])


_scmesh = plsc.VectorSubcoreMesh(core_axis_name="c", subcore_axis_name="s")

_sc_router_call = functools.partial(
    pl.kernel,
    mesh=_scmesh,
    out_type=[
        jax.ShapeDtypeStruct((_NC * _E,), jnp.float32),
        jax.ShapeDtypeStruct((_NC * _TOPK,), jnp.float32),
        jax.ShapeDtypeStruct((_NC * _TOPK,), jnp.int32),
        jax.ShapeDtypeStruct((_NTILES, _L), jnp.float32),
        jax.ShapeDtypeStruct((_NTILES, _L), jnp.float32),
    ],
    scratch_types=[
        pltpu.VMEM((_E, _TPT), jnp.float32),
        pltpu.VMEM((_TPT * _E,), jnp.float32),
        pltpu.VMEM((_TPT * _TOPK,), jnp.float32),
        pltpu.VMEM((_TPT * _TOPK,), jnp.int32),
        pltpu.VMEM((_L,), jnp.float32),
        pltpu.VMEM((_L,), jnp.float32),
    ],
    compiler_params=pltpu.CompilerParams(needs_layout_passes=False),
)(_sc_router)


def kernel(hidden_states, W):
    x = hidden_states.reshape(_N, _H)

    def matmul_chunk(c):
        return pl.pallas_call(
            _gate_kernel,
            grid=(_GRID,),
            in_specs=[
                pl.BlockSpec((_BLOCK_T, _H), lambda i, c=c: (c * _GRID + i,
                                                             0)),
                pl.BlockSpec((_E, _H), lambda i: (0, 0)),
            ],
            out_specs=pl.BlockSpec((_E, _BLOCK_T), lambda i: (0, i)),
            out_shape=jax.ShapeDtypeStruct((_E, _NC), jnp.float32),
            compiler_params=pltpu.CompilerParams(
                dimension_semantics=("arbitrary",)),
        )(x, W)

    lgts = [matmul_chunk(c) for c in range(_NCHUNK)]
    routed = [_sc_router_call(lgt) for lgt in lgts]

    probs_f = jnp.concatenate([r[0] for r in routed])
    wts_f = jnp.concatenate([r[1] for r in routed])
    idx_f = jnp.concatenate([r[2] for r in routed])
    ew = sum(jnp.sum(r[3], axis=0) for r in routed)
    ec = sum(jnp.sum(r[4], axis=0) for r in routed)
    expected = _N * _TOPK / _E
    loss = jnp.sum(ew * ec) / (expected * expected)

    return (wts_f.reshape(_B, _S, _TOPK), idx_f.reshape(_B, _S, _TOPK),
            probs_f.reshape(_B, _S, _E), loss)


# speedup vs baseline: 1.0819x; 1.0819x over previous
import functools

import jax
import jax.numpy as jnp
from jax import lax
from jax.experimental import pallas as pl
from jax.experimental.pallas import tpu as pltpu
from jax.experimental.pallas import tpu_sc as plsc

_B, _S, _H = 4, 8192, 768
_E, _TOPK = 8, 2
_N = _B * _S

_BLOCK_T = 2048
_GRID = _N // _BLOCK_T

_NTILES = 32
_TPT = _N // _NTILES
_L = 16


def _gate_kernel(x_ref, w_ref, out_ref):
    out_ref[...] = jax.lax.dot_general(
        w_ref[...], x_ref[...], (((1,), (1,)), ((), ())),
        preferred_element_type=jnp.float32)


def _sc_router(lgt_hbm, probs_hbm, wts_hbm, idx_hbm, ewp_hbm, ecp_hbm,
               lg_v, pb_v, wb_v, ib_v, ew_v, ec_v):
    wid = lax.axis_index("s") * 2 + lax.axis_index("c")
    base = wid * _TPT
    pltpu.sync_copy(lgt_hbm.at[:, pl.ds(base, _TPT)], lg_v)

    zero = jnp.zeros((_L,), jnp.float32)

    def chunk(c, carry):
        ews, ecs = carry
        off = c * _L
        ls = [lg_v[e, pl.ds(off, _L)] for e in range(_E)]
        m = ls[0]
        for e in range(1, _E):
            m = jnp.maximum(m, ls[e])
        exs = [jnp.exp(l - m) for l in ls]
        s = exs[0]
        for e in range(1, _E):
            s = s + exs[e]
        inv = 1.0 / s
        ps = [ex * inv for ex in exs]

        best = ps[0]
        bidx = jnp.zeros((_L,), jnp.int32)
        for e in range(1, _E):
            upd = ps[e] > best
            best = jnp.where(upd, ps[e], best)
            bidx = jnp.where(upd, e, bidx)
        second = jnp.full((_L,), -1.0, jnp.float32)
        sidx = jnp.zeros((_L,), jnp.int32)
        for e in range(_E):
            upd = (ps[e] > second) & (bidx != e)
            second = jnp.where(upd, ps[e], second)
            sidx = jnp.where(upd, e, sidx)

        inv2 = 1.0 / (best + second + 1e-8)
        w1 = best * inv2
        w2 = second * inv2

        ii = lax.iota(jnp.int32, _L)
        t8 = (off + ii) * _E
        for e in range(_E):
            plsc.store_scatter(pb_v, [t8 + e], ps[e])
        t2 = (off + ii) * _TOPK
        plsc.store_scatter(wb_v, [t2], w1)
        plsc.store_scatter(wb_v, [t2 + 1], w2)
        plsc.store_scatter(ib_v, [t2], bidx)
        plsc.store_scatter(ib_v, [t2 + 1], sidx)

        new_ews = tuple(
            ews[e] + jnp.where(bidx == e, w1, zero)
            + jnp.where(sidx == e, w2, zero) for e in range(_E))
        new_ecs = tuple(
            ecs[e] + (bidx == e).astype(jnp.float32)
            + (sidx == e).astype(jnp.float32) for e in range(_E))
        return new_ews, new_ecs

    init = (tuple(zero for _ in range(_E)), tuple(zero for _ in range(_E)))
    ews, ecs = lax.fori_loop(0, _TPT // _L, chunk, init)

    ii = lax.iota(jnp.int32, _L)
    ew_row = zero
    ec_row = zero
    for e in range(_E):
        ew_row = jnp.where(ii == e, jnp.sum(ews[e]), ew_row)
        ec_row = jnp.where(ii == e, jnp.sum(ecs[e]), ec_row)
    ew_v[...] = ew_row
    ec_v[...] = ec_row

    pltpu.sync_copy(pb_v, probs_hbm.at[pl.ds(base * _E, _TPT * _E)])
    pltpu.sync_copy(wb_v, wts_hbm.at[pl.ds(base * _TOPK, _TPT * _TOPK)])
    pltpu.sync_copy(ib_v, idx_hbm.at[pl.ds(base * _TOPK, _TPT * _TOPK)])
    pltpu.sync_copy(ew_v, ewp_hbm.at[wid])
    pltpu.sync_copy(ec_v, ecp_hbm.at[wid])


_scmesh = plsc.VectorSubcoreMesh(core_axis_name="c", subcore_axis_name="s")

_sc_router_call = functools.partial(
    pl.kernel,
    mesh=_scmesh,
    out_type=[
        jax.ShapeDtypeStruct((_N * _E,), jnp.float32),
        jax.ShapeDtypeStruct((_N * _TOPK,), jnp.float32),
        jax.ShapeDtypeStruct((_N * _TOPK,), jnp.int32),
        jax.ShapeDtypeStruct((_NTILES, _L), jnp.float32),
        jax.ShapeDtypeStruct((_NTILES, _L), jnp.float32),
    ],
    scratch_types=[
        pltpu.VMEM((_E, _TPT), jnp.float32),
        pltpu.VMEM((_TPT * _E,), jnp.float32),
        pltpu.VMEM((_TPT * _TOPK,), jnp.float32),
        pltpu.VMEM((_TPT * _TOPK,), jnp.int32),
        pltpu.VMEM((_L,), jnp.float32),
        pltpu.VMEM((_L,), jnp.float32),
    ],
    compiler_params=pltpu.CompilerParams(needs_layout_passes=False),
)(_sc_router)


def kernel(hidden_states, W):
    x = hidden_states.reshape(_N, _H)
    lgt = pl.pallas_call(
        _gate_kernel,
        grid=(_GRID,),
        in_specs=[
            pl.BlockSpec((_BLOCK_T, _H), lambda i: (i, 0)),
            pl.BlockSpec((_E, _H), lambda i: (0, 0)),
        ],
        out_specs=pl.BlockSpec((_E, _BLOCK_T), lambda i: (0, i)),
        out_shape=jax.ShapeDtypeStruct((_E, _N), jnp.float32),
        compiler_params=pltpu.CompilerParams(
            dimension_semantics=("arbitrary",)),
    )(x, W)

    probs_f, wts_f, idx_f, ewp, ecp = _sc_router_call(lgt)

    ew = jnp.sum(ewp, axis=0)
    ec = jnp.sum(ecp, axis=0)
    expected = _N * _TOPK / _E
    loss = jnp.sum(ew * ec) / (expected * expected)

    return (wts_f.reshape(_B, _S, _TOPK), idx_f.reshape(_B, _S, _TOPK),
            probs_f.reshape(_B, _S, _E), loss)


# speedup vs baseline: 2.4499x; 2.2643x over previous
import functools

import jax
import jax.numpy as jnp
from jax import lax
from jax.experimental import pallas as pl
from jax.experimental.pallas import tpu as pltpu
from jax.experimental.pallas import tpu_sc as plsc

_B, _S, _H = 4, 8192, 768
_E, _TOPK = 8, 2
_N = _B * _S

_BLOCK_T = 2048
_GRID = _N // _BLOCK_T

_NTILES = 32
_TPT = _N // _NTILES
_L = 16


def _gate_kernel(x_ref, w_ref, out_ref):
    out_ref[...] = jax.lax.dot_general(
        w_ref[...], x_ref[...], (((1,), (1,)), ((), ())),
        preferred_element_type=jnp.float32)


def _sc_router(lgt_hbm, probs_hbm, wts_hbm, idx_hbm, ewp_hbm, ecp_hbm,
               lg_v, pb_v, wb_v, ib_v, ew_v, ec_v):
    wid = lax.axis_index("s") * 2 + lax.axis_index("c")
    base = wid * _TPT
    pltpu.sync_copy(lgt_hbm.at[:, pl.ds(base, _TPT)], lg_v)

    zero = jnp.zeros((_L,), jnp.float32)

    def chunk(c, carry):
        ews, ecs = carry
        off = c * _L
        ls = [lg_v[e, pl.ds(off, _L)] for e in range(_E)]
        m = ls[0]
        for e in range(1, _E):
            m = jnp.maximum(m, ls[e])
        exs = [jnp.exp(l - m) for l in ls]
        s = exs[0]
        for e in range(1, _E):
            s = s + exs[e]
        inv = 1.0 / s
        ps = [ex * inv for ex in exs]

        best = ps[0]
        bidx = jnp.zeros((_L,), jnp.int32)
        for e in range(1, _E):
            upd = ps[e] > best
            best = jnp.where(upd, ps[e], best)
            bidx = jnp.where(upd, e, bidx)
        second = jnp.full((_L,), -1.0, jnp.float32)
        sidx = jnp.zeros((_L,), jnp.int32)
        for e in range(_E):
            upd = (ps[e] > second) & (bidx != e)
            second = jnp.where(upd, ps[e], second)
            sidx = jnp.where(upd, e, sidx)

        inv2 = 1.0 / (best + second + 1e-8)
        w1 = best * inv2
        w2 = second * inv2

        for e in range(_E):
            pb_v[e, pl.ds(off, _L)] = ps[e]
        wb_v[0, pl.ds(off, _L)] = w1
        wb_v[1, pl.ds(off, _L)] = w2
        ib_v[0, pl.ds(off, _L)] = bidx
        ib_v[1, pl.ds(off, _L)] = sidx

        new_ews = tuple(
            ews[e] + jnp.where(bidx == e, w1, zero)
            + jnp.where(sidx == e, w2, zero) for e in range(_E))
        new_ecs = tuple(
            ecs[e] + (bidx == e).astype(jnp.float32)
            + (sidx == e).astype(jnp.float32) for e in range(_E))
        return new_ews, new_ecs

    init = (tuple(zero for _ in range(_E)), tuple(zero for _ in range(_E)))
    ews, ecs = lax.fori_loop(0, _TPT // _L, chunk, init)

    ii = lax.iota(jnp.int32, _L)
    ew_row = zero
    ec_row = zero
    for e in range(_E):
        ew_row = jnp.where(ii == e, jnp.sum(ews[e]), ew_row)
        ec_row = jnp.where(ii == e, jnp.sum(ecs[e]), ec_row)
    ew_v[...] = ew_row
    ec_v[...] = ec_row

    pltpu.sync_copy(pb_v, probs_hbm.at[:, pl.ds(base, _TPT)])
    pltpu.sync_copy(wb_v, wts_hbm.at[:, pl.ds(base, _TPT)])
    pltpu.sync_copy(ib_v, idx_hbm.at[:, pl.ds(base, _TPT)])
    pltpu.sync_copy(ew_v, ewp_hbm.at[wid])
    pltpu.sync_copy(ec_v, ecp_hbm.at[wid])


_scmesh = plsc.VectorSubcoreMesh(core_axis_name="c", subcore_axis_name="s")

_sc_router_call = functools.partial(
    pl.kernel,
    mesh=_scmesh,
    out_type=[
        jax.ShapeDtypeStruct((_E, _N), jnp.float32),
        jax.ShapeDtypeStruct((_TOPK, _N), jnp.float32),
        jax.ShapeDtypeStruct((_TOPK, _N), jnp.int32),
        jax.ShapeDtypeStruct((_NTILES, _L), jnp.float32),
        jax.ShapeDtypeStruct((_NTILES, _L), jnp.float32),
    ],
    scratch_types=[
        pltpu.VMEM((_E, _TPT), jnp.float32),
        pltpu.VMEM((_E, _TPT), jnp.float32),
        pltpu.VMEM((_TOPK, _TPT), jnp.float32),
        pltpu.VMEM((_TOPK, _TPT), jnp.int32),
        pltpu.VMEM((_L,), jnp.float32),
        pltpu.VMEM((_L,), jnp.float32),
    ],
    compiler_params=pltpu.CompilerParams(needs_layout_passes=False),
)(_sc_router)


def kernel(hidden_states, W):
    x = hidden_states.reshape(_N, _H)
    lgt = pl.pallas_call(
        _gate_kernel,
        grid=(_GRID,),
        in_specs=[
            pl.BlockSpec((_BLOCK_T, _H), lambda i: (i, 0)),
            pl.BlockSpec((_E, _H), lambda i: (0, 0)),
        ],
        out_specs=pl.BlockSpec((_E, _BLOCK_T), lambda i: (0, i)),
        out_shape=jax.ShapeDtypeStruct((_E, _N), jnp.float32),
        compiler_params=pltpu.CompilerParams(
            dimension_semantics=("arbitrary",)),
    )(x, W)

    probs_f, wts_f, idx_f, ewp, ecp = _sc_router_call(lgt)

    ew = jnp.sum(ewp, axis=0)
    ec = jnp.sum(ecp, axis=0)
    expected = _N * _TOPK / _E
    loss = jnp.sum(ew * ec) / (expected * expected)

    return (wts_f.T.reshape(_B, _S, _TOPK), idx_f.T.reshape(_B, _S, _TOPK),
            probs_f.T.reshape(_B, _S, _E), loss)


# speedup vs baseline: 2.4648x; 1.0061x over previous
import functools

import jax
import jax.numpy as jnp
from jax import lax
from jax.experimental import pallas as pl
from jax.experimental.pallas import tpu as pltpu
from jax.experimental.pallas import tpu_sc as plsc

_B, _S, _H = 4, 8192, 768
_E, _TOPK = 8, 2
_N = _B * _S

_BLOCK_T = 4096
_GRID = _N // _BLOCK_T

_NTILES = 32
_TPT = _N // _NTILES
_L = 16


def _gate_kernel(x_ref, w_ref, out_ref):
    out_ref[...] = jax.lax.dot_general(
        w_ref[...], x_ref[...], (((1,), (1,)), ((), ())),
        preferred_element_type=jnp.float32)


def _sc_router(lgt_hbm, probs_hbm, wts_hbm, idx_hbm, ewp_hbm, ecp_hbm,
               lg_v, pb_v, wb_v, ib_v, ew_v, ec_v):
    wid = lax.axis_index("s") * 2 + lax.axis_index("c")
    base = wid * _TPT
    pltpu.sync_copy(lgt_hbm.at[:, pl.ds(base, _TPT)], lg_v)

    zero = jnp.zeros((_L,), jnp.float32)

    def chunk(c, carry):
        ews, ecs = carry
        off = c * _L
        ls = [lg_v[e, pl.ds(off, _L)] for e in range(_E)]
        m = ls[0]
        for e in range(1, _E):
            m = jnp.maximum(m, ls[e])
        exs = [jnp.exp(l - m) for l in ls]
        s = exs[0]
        for e in range(1, _E):
            s = s + exs[e]
        inv = 1.0 / s
        ps = [ex * inv for ex in exs]

        best = ps[0]
        bidx = jnp.zeros((_L,), jnp.int32)
        for e in range(1, _E):
            upd = ps[e] > best
            best = jnp.where(upd, ps[e], best)
            bidx = jnp.where(upd, e, bidx)
        second = jnp.full((_L,), -1.0, jnp.float32)
        sidx = jnp.zeros((_L,), jnp.int32)
        for e in range(_E):
            upd = (ps[e] > second) & (bidx != e)
            second = jnp.where(upd, ps[e], second)
            sidx = jnp.where(upd, e, sidx)

        inv2 = 1.0 / (best + second + 1e-8)
        w1 = best * inv2
        w2 = second * inv2

        for e in range(_E):
            pb_v[e, pl.ds(off, _L)] = ps[e]
        wb_v[0, pl.ds(off, _L)] = w1
        wb_v[1, pl.ds(off, _L)] = w2
        ib_v[0, pl.ds(off, _L)] = bidx
        ib_v[1, pl.ds(off, _L)] = sidx

        new_ews = tuple(
            ews[e] + jnp.where(bidx == e, w1, zero)
            + jnp.where(sidx == e, w2, zero) for e in range(_E))
        new_ecs = tuple(
            ecs[e] + (bidx == e).astype(jnp.float32)
            + (sidx == e).astype(jnp.float32) for e in range(_E))
        return new_ews, new_ecs

    init = (tuple(zero for _ in range(_E)), tuple(zero for _ in range(_E)))
    ews, ecs = lax.fori_loop(0, _TPT // _L, chunk, init)

    ii = lax.iota(jnp.int32, _L)
    ew_row = zero
    ec_row = zero
    for e in range(_E):
        ew_row = jnp.where(ii == e, jnp.sum(ews[e]), ew_row)
        ec_row = jnp.where(ii == e, jnp.sum(ecs[e]), ec_row)
    ew_v[...] = ew_row
    ec_v[...] = ec_row

    pltpu.sync_copy(pb_v, probs_hbm.at[:, pl.ds(base, _TPT)])
    pltpu.sync_copy(wb_v, wts_hbm.at[:, pl.ds(base, _TPT)])
    pltpu.sync_copy(ib_v, idx_hbm.at[:, pl.ds(base, _TPT)])
    pltpu.sync_copy(ew_v, ewp_hbm.at[wid])
    pltpu.sync_copy(ec_v, ecp_hbm.at[wid])


_scmesh = plsc.VectorSubcoreMesh(core_axis_name="c", subcore_axis_name="s")

_sc_router_call = functools.partial(
    pl.kernel,
    mesh=_scmesh,
    out_type=[
        jax.ShapeDtypeStruct((_E, _N), jnp.float32),
        jax.ShapeDtypeStruct((_TOPK, _N), jnp.float32),
        jax.ShapeDtypeStruct((_TOPK, _N), jnp.int32),
        jax.ShapeDtypeStruct((_NTILES, _L), jnp.float32),
        jax.ShapeDtypeStruct((_NTILES, _L), jnp.float32),
    ],
    scratch_types=[
        pltpu.VMEM((_E, _TPT), jnp.float32),
        pltpu.VMEM((_E, _TPT), jnp.float32),
        pltpu.VMEM((_TOPK, _TPT), jnp.float32),
        pltpu.VMEM((_TOPK, _TPT), jnp.int32),
        pltpu.VMEM((_L,), jnp.float32),
        pltpu.VMEM((_L,), jnp.float32),
    ],
    compiler_params=pltpu.CompilerParams(needs_layout_passes=False),
)(_sc_router)


def kernel(hidden_states, W):
    x = hidden_states.reshape(_N, _H)
    lgt = pl.pallas_call(
        _gate_kernel,
        grid=(_GRID,),
        in_specs=[
            pl.BlockSpec((_BLOCK_T, _H), lambda i: (i, 0)),
            pl.BlockSpec((_E, _H), lambda i: (0, 0)),
        ],
        out_specs=pl.BlockSpec((_E, _BLOCK_T), lambda i: (0, i)),
        out_shape=jax.ShapeDtypeStruct((_E, _N), jnp.float32),
        compiler_params=pltpu.CompilerParams(
            dimension_semantics=("arbitrary",)),
    )(x, W)

    probs_f, wts_f, idx_f, ewp, ecp = _sc_router_call(lgt)

    ew = jnp.sum(ewp, axis=0)
    ec = jnp.sum(ecp, axis=0)
    expected = _N * _TOPK / _E
    loss = jnp.sum(ew * ec) / (expected * expected)

    return (wts_f.T.reshape(_B, _S, _TOPK), idx_f.T.reshape(_B, _S, _TOPK),
            probs_f.T.reshape(_B, _S, _E), loss)
